# baseline (speedup 1.0000x reference)
import functools

import jax
import jax.numpy as jnp
from jax import lax
from jax.experimental import pallas as pl
from jax.experimental.pallas import tpu as pltpu
from jax.experimental.pallas import tpu_sc as plsc

CHAR = 1000000
DIM = 64
PADW = 128
ROWS = 16384
COLS = 50
CP2 = COLS // 2

NC = 2
NS = 16
NW = NC * NS

R_PER_W = ROWS // NW
RX = 4
N_CHUNKS = R_PER_W // RX
NB = 4

TBN = 16384
TGRID = -(-CHAR // TBN)

BI = 512
OGRID = ROWS // BI


def _transpose_pad_body(lutT_ref, out_ref):
    out_ref[:, :DIM] = jnp.swapaxes(lutT_ref[...], 0, 1)


def _out_transpose_body(lin_ref, out_ref):
    a = jnp.reshape(lin_ref[...], (BI, CP2, PADW))
    for jp in range(CP2):
        out_ref[jp] = jnp.swapaxes(a[:, jp, :], 0, 1)


def _emb_body(x_hbm, lut_hbm, out_hbm, idx_v, rb0, rb1, rb2, rb3,
              sg0, sg1, sg2, sg3, so0, so1, so2, so3):
    rbufs = (rb0, rb1, rb2, rb3)
    gsems = (sg0, sg1, sg2, sg3)
    osems = (so0, so1, so2, so3)
    wid = lax.axis_index("s") * NC + lax.axis_index("c")
    base = wid * R_PER_W

    pltpu.sync_copy(x_hbm.at[pl.ds(base, R_PER_W)], idx_v)

    def fire_gathers(i, b):
        for u in range(RX):
            pltpu.async_copy(
                lut_hbm.at[idx_v.at[i * RX + u]],
                rbufs[b].at[u],
                gsems[b],
            )

    def wait_gathers(b):
        pltpu.make_async_copy(lut_hbm.at[pl.ds(0, RX * COLS)], rbufs[b], gsems[b]).wait()

    def fire_out(i, b):
        pltpu.async_copy(
            rbufs[b].at[:, :, pl.ds(0, DIM)],
            out_hbm.at[pl.ds(base + i * RX, RX)],
            osems[b],
        )

    def wait_out(b):
        pltpu.make_async_copy(
            rbufs[b].at[:, :, pl.ds(0, DIM)],
            out_hbm.at[pl.ds(0, RX)],
            osems[b],
        ).wait()

    for b in range(NB):
        fire_gathers(b, b)

    def loop_body(g, carry):
        for b in range(NB):
            i = g * NB + b
            wait_gathers(b)
            fire_out(i, b)
            nxt = i + NB

            @pl.when(nxt < N_CHUNKS)
            def _():
                wait_out(b)
                fire_gathers(nxt, b)

        return carry

    lax.fori_loop(0, N_CHUNKS // NB, loop_body, 0)
    for b in range(NB):
        wait_out(b)


@functools.partial(jax.jit, static_argnames=())
def kernel(x, lut):
    lut_t = jnp.swapaxes(lut, 0, 1)
    lut_p = pl.pallas_call(
        _transpose_pad_body,
        out_shape=jax.ShapeDtypeStruct((CHAR, PADW), jnp.float32),
        grid=(TGRID,),
        in_specs=[pl.BlockSpec((DIM, TBN), lambda i: (0, i))],
        out_specs=pl.BlockSpec((TBN, PADW), lambda i: (i, 0)),
    )(lut_t)

    mesh = plsc.VectorSubcoreMesh(core_axis_name="c", subcore_axis_name="s")
    out_sc = pl.kernel(
        _emb_body,
        out_type=jax.ShapeDtypeStruct((ROWS, COLS, DIM), jnp.float32),
        mesh=mesh,
        compiler_params=pltpu.CompilerParams(use_tc_tiling_on_sc=False),
        scratch_types=[
            pltpu.VMEM((R_PER_W, COLS), jnp.int32),
            pltpu.VMEM((RX, COLS, PADW), jnp.float32),
            pltpu.VMEM((RX, COLS, PADW), jnp.float32),
            pltpu.VMEM((RX, COLS, PADW), jnp.float32),
            pltpu.VMEM((RX, COLS, PADW), jnp.float32),
            pltpu.SemaphoreType.DMA,
            pltpu.SemaphoreType.DMA,
            pltpu.SemaphoreType.DMA,
            pltpu.SemaphoreType.DMA,
            pltpu.SemaphoreType.DMA,
            pltpu.SemaphoreType.DMA,
            pltpu.SemaphoreType.DMA,
            pltpu.SemaphoreType.DMA,
        ],
    )(x.astype(jnp.int32), lut_p)

    pairs = jnp.reshape(out_sc, (ROWS * CP2, PADW))
    p2 = pl.pallas_call(
        _out_transpose_body,
        out_shape=jax.ShapeDtypeStruct((CP2, PADW, ROWS), jnp.float32),
        grid=(OGRID,),
        in_specs=[pl.BlockSpec((BI * CP2, PADW), lambda i: (i, 0))],
        out_specs=pl.BlockSpec((CP2, PADW, BI), lambda i: (0, 0, i)),
    )(pairs)
    return jnp.transpose(p2, (2, 0, 1)).reshape(ROWS, COLS, DIM)


# speedup vs baseline: 1.0097x; 1.0097x over previous
import functools

import jax
import jax.numpy as jnp
from jax import lax
from jax.experimental import pallas as pl
from jax.experimental.pallas import tpu as pltpu
from jax.experimental.pallas import tpu_sc as plsc

CHAR = 1000000
DIM = 64
PADW = 128
ROWS = 16384
COLS = 50
CP2 = COLS // 2

NC = 2
NS = 16
NW = NC * NS

R_PER_W = ROWS // NW
RX = 8
N_CHUNKS = R_PER_W // RX
NB = 2

TBN = 16384
TGRID = -(-CHAR // TBN)

BI = 512
OGRID = ROWS // BI


def _transpose_pad_body(lutT_ref, out_ref):
    out_ref[:, :DIM] = jnp.swapaxes(lutT_ref[...], 0, 1)


def _out_transpose_body(lin_ref, out_ref):
    a = jnp.reshape(lin_ref[...], (BI, CP2, PADW))
    for jp in range(CP2):
        out_ref[jp] = jnp.swapaxes(a[:, jp, :], 0, 1)


def _emb_body(x_hbm, lut_hbm, out_hbm, idx_v, rb0, rb1, sg0, sg1, so0, so1):
    rbufs = (rb0, rb1)
    gsems = (sg0, sg1)
    osems = (so0, so1)
    wid = lax.axis_index("s") * NC + lax.axis_index("c")
    base = wid * R_PER_W

    pltpu.sync_copy(x_hbm.at[pl.ds(base, R_PER_W)], idx_v)

    def fire_gathers(i, b):
        for u in range(RX):
            pltpu.async_copy(
                lut_hbm.at[idx_v.at[i * RX + u]],
                rbufs[b].at[u],
                gsems[b],
            )

    def wait_gathers(b):
        pltpu.make_async_copy(lut_hbm.at[pl.ds(0, RX * COLS)], rbufs[b], gsems[b]).wait()

    def fire_out(i, b):
        pltpu.async_copy(
            rbufs[b].at[:, :, pl.ds(0, DIM)],
            out_hbm.at[pl.ds(base + i * RX, RX)],
            osems[b],
        )

    def wait_out(b):
        pltpu.make_async_copy(
            rbufs[b].at[:, :, pl.ds(0, DIM)],
            out_hbm.at[pl.ds(0, RX)],
            osems[b],
        ).wait()

    for b in range(NB):
        fire_gathers(b, b)

    def loop_body(g, carry):
        for b in range(NB):
            i = g * NB + b
            wait_gathers(b)
            fire_out(i, b)
            nxt = i + NB

            @pl.when(nxt < N_CHUNKS)
            def _():
                wait_out(b)
                fire_gathers(nxt, b)

        return carry

    lax.fori_loop(0, N_CHUNKS // NB, loop_body, 0)
    for b in range(NB):
        wait_out(b)


@functools.partial(jax.jit, static_argnames=())
def kernel(x, lut):
    lut_t = jnp.swapaxes(lut, 0, 1)
    lut_p = pl.pallas_call(
        _transpose_pad_body,
        out_shape=jax.ShapeDtypeStruct((CHAR, PADW), jnp.float32),
        grid=(TGRID,),
        in_specs=[pl.BlockSpec((DIM, TBN), lambda i: (0, i))],
        out_specs=pl.BlockSpec((TBN, PADW), lambda i: (i, 0)),
    )(lut_t)

    mesh = plsc.VectorSubcoreMesh(core_axis_name="c", subcore_axis_name="s")
    out_sc = pl.kernel(
        _emb_body,
        out_type=jax.ShapeDtypeStruct((ROWS, COLS, DIM), jnp.float32),
        mesh=mesh,
        compiler_params=pltpu.CompilerParams(use_tc_tiling_on_sc=False),
        scratch_types=[
            pltpu.VMEM((R_PER_W, COLS), jnp.int32),
            pltpu.VMEM((RX, COLS, PADW), jnp.float32),
            pltpu.VMEM((RX, COLS, PADW), jnp.float32),
            pltpu.SemaphoreType.DMA,
            pltpu.SemaphoreType.DMA,
            pltpu.SemaphoreType.DMA,
            pltpu.SemaphoreType.DMA,
        ],
    )(x.astype(jnp.int32), lut_p)

    pairs = jnp.reshape(out_sc, (ROWS * CP2, PADW))
    p2 = pl.pallas_call(
        _out_transpose_body,
        out_shape=jax.ShapeDtypeStruct((CP2, PADW, ROWS), jnp.float32),
        grid=(OGRID,),
        in_specs=[pl.BlockSpec((BI * CP2, PADW), lambda i: (i, 0))],
        out_specs=pl.BlockSpec((CP2, PADW, BI), lambda i: (0, 0, i)),
    )(pairs)
    return jnp.transpose(p2, (2, 0, 1)).reshape(ROWS, COLS, DIM)
